# baseline (speedup 1.0000x reference)
import jax
import jax.numpy as jnp
from jax import lax
from jax.experimental import pallas as pl
from jax.experimental.pallas import tpu as pltpu

_B, _C, _H, _W = 128, 3, 32, 32
_OC, _KH, _KW = 64, 3, 3
_S = _H * _W
_K = _OC * _S
_KROWS = 32
_G = 64
_KT = 8


def _conv_kernel(a_ref, wc_ref, o_ref):
    wc = wc_ref[...]
    for i in range(_G):
        o_ref[i] = jnp.dot(wc, a_ref[i],
                           preferred_element_type=jnp.float32
                           ).astype(o_ref.dtype)


def _linear_kernel(h_ref, wmu_ref, wxi_ref, o_ref, acc_ref):
    k = pl.program_id(1)

    @pl.when(k == 0)
    def _():
        acc_ref[...] = jnp.zeros_like(acc_ref)

    w = wmu_ref[...] + wxi_ref[...]
    acc_ref[...] += lax.dot_general(
        h_ref[...], w, (((1,), (1,)), ((), ())),
        preferred_element_type=jnp.float32)

    @pl.when(k == _KT - 1)
    def _():
        o_ref[0] = acc_ref[...]


def _build_patches(x):
    f32 = jnp.float32
    xv = jnp.pad(x.astype(f32), ((0, 0), (0, 0), (1, 1), (0, 0)))
    pieces = []
    for kh in range(_KH):
        rows = xv[:, :, kh:kh + _H, :]
        for kw in range(_KW):
            if kw == 0:
                sh = jnp.pad(rows, ((0, 0), (0, 0), (0, 0), (1, 0)))[..., :_W]
            elif kw == 1:
                sh = rows
            else:
                sh = jnp.pad(rows, ((0, 0), (0, 0), (0, 0), (0, 1)))[..., 1:]
            pieces.append(sh)
    a = jnp.stack(pieces, axis=1).reshape(_B, 9 * _C, _S)
    pad = jnp.concatenate(
        [jnp.ones((_B, 1, _S), f32), jnp.zeros((_B, _KROWS - 9 * _C - 1, _S), f32)],
        axis=1)
    return jnp.concatenate([a, pad], axis=1)


def kernel(x, conv_mu, conv_b_mu, conv_stdev_xi, conv_b_stdev_xi,
           lin_mu, lin_b_mu, lin_stdev_xi, lin_b_stdev_xi):
    f32 = jnp.float32
    a = _build_patches(x)

    wc = (conv_mu + conv_stdev_xi).astype(f32)
    wc = wc.transpose(0, 2, 3, 1).reshape(_OC, 9 * _C)
    cb = (conv_b_mu + conv_b_stdev_xi).astype(f32).reshape(_OC, 1)
    wc = jnp.concatenate(
        [wc, cb, jnp.zeros((_OC, _KROWS - 9 * _C - 1), f32)], axis=1)

    h = pl.pallas_call(
        _conv_kernel,
        out_shape=jax.ShapeDtypeStruct((_B, _OC, _S), f32),
        grid=(_B // _G,),
        in_specs=[
            pl.BlockSpec((_G, _KROWS, _S), lambda g: (g, 0, 0)),
            pl.BlockSpec((_OC, _KROWS), lambda g: (0, 0)),
        ],
        out_specs=pl.BlockSpec((_G, _OC, _S), lambda g: (g, 0, 0)),
        compiler_params=pltpu.CompilerParams(
            dimension_semantics=("parallel",),
            vmem_limit_bytes=48 * 1024 * 1024),
        cost_estimate=pl.CostEstimate(
            flops=2 * _B * _OC * _KROWS * _S, transcendentals=0,
            bytes_accessed=4 * (_B * _KROWS * _S + _B * _OC * _S)),
    )(a, wc)

    hf = h.reshape(_B, _K)
    tk = _K // (2 * _KT)

    part = pl.pallas_call(
        _linear_kernel,
        out_shape=jax.ShapeDtypeStruct((2, _B, 128), f32),
        grid=(2, _KT),
        in_specs=[
            pl.BlockSpec((_B, tk), lambda n, k: (0, n * _KT + k)),
            pl.BlockSpec((128, tk), lambda n, k: (0, n * _KT + k)),
            pl.BlockSpec((128, tk), lambda n, k: (0, n * _KT + k)),
        ],
        out_specs=pl.BlockSpec((1, _B, 128), lambda n, k: (n, 0, 0)),
        scratch_shapes=[pltpu.VMEM((_B, 128), f32)],
        compiler_params=pltpu.CompilerParams(
            dimension_semantics=("parallel", "arbitrary"),
            vmem_limit_bytes=48 * 1024 * 1024),
        cost_estimate=pl.CostEstimate(
            flops=2 * _B * 128 * _K + _K * 128, transcendentals=0,
            bytes_accessed=4 * (_B * _K + 2 * 128 * _K + 2 * _B * 128)),
    )(hf, lin_mu, lin_stdev_xi)

    bias = (lin_b_mu + lin_b_stdev_xi).reshape(1, 128)
    return part[0] + part[1] + bias


# speedup vs baseline: 1.2157x; 1.2157x over previous
import jax
import jax.numpy as jnp
from jax import lax
from jax.experimental import pallas as pl
from jax.experimental.pallas import tpu as pltpu

_B, _C, _H, _W = 128, 3, 32, 32
_OC, _KH, _KW = 64, 3, 3
_S = _H * _W
_K = _OC * _S
_KROWS = 32
_G = 32
_KT = 4
_OCC = _OC // (2 * _KT)


def _conv_kernel(a_ref, wc_ref, o_ref):
    wc = wc_ref[...]
    for i in range(_G):
        o_ref[i] = jnp.dot(wc, a_ref[i],
                           preferred_element_type=jnp.float32
                           ).astype(o_ref.dtype)


def _linear_kernel(h_ref, wmu_ref, wxi_ref, o_ref, acc_ref):
    k = pl.program_id(1)

    @pl.when(k == 0)
    def _():
        acc_ref[...] = jnp.zeros_like(acc_ref)

    w = wmu_ref[...] + wxi_ref[...]
    acc = acc_ref[...]
    for c in range(_OCC):
        acc += lax.dot_general(
            h_ref[:, c, :], w[:, c * _S:(c + 1) * _S],
            (((1,), (1,)), ((), ())),
            preferred_element_type=jnp.float32)
    acc_ref[...] = acc

    @pl.when(k == _KT - 1)
    def _():
        o_ref[0] = acc_ref[...]


def _build_patches(x):
    f32 = jnp.float32
    xv = jnp.pad(x.astype(f32), ((0, 0), (0, 0), (1, 1), (0, 0)))
    pieces = []
    for kh in range(_KH):
        rows = xv[:, :, kh:kh + _H, :]
        for kw in range(_KW):
            if kw == 0:
                sh = jnp.pad(rows, ((0, 0), (0, 0), (0, 0), (1, 0)))[..., :_W]
            elif kw == 1:
                sh = rows
            else:
                sh = jnp.pad(rows, ((0, 0), (0, 0), (0, 0), (0, 1)))[..., 1:]
            pieces.append(sh)
    a = jnp.stack(pieces, axis=1).reshape(_B, 9 * _C, _S)
    pad = jnp.concatenate(
        [jnp.ones((_B, 1, _S), f32), jnp.zeros((_B, _KROWS - 9 * _C - 1, _S), f32)],
        axis=1)
    return jnp.concatenate([a, pad], axis=1)


def kernel(x, conv_mu, conv_b_mu, conv_stdev_xi, conv_b_stdev_xi,
           lin_mu, lin_b_mu, lin_stdev_xi, lin_b_stdev_xi):
    f32 = jnp.float32
    a = _build_patches(x)

    wc = (conv_mu + conv_stdev_xi).astype(f32)
    wc = wc.transpose(0, 2, 3, 1).reshape(_OC, 9 * _C)
    cb = (conv_b_mu + conv_b_stdev_xi).astype(f32).reshape(_OC, 1)
    wc = jnp.concatenate(
        [wc, cb, jnp.zeros((_OC, _KROWS - 9 * _C - 1), f32)], axis=1)

    h = pl.pallas_call(
        _conv_kernel,
        out_shape=jax.ShapeDtypeStruct((_B, _OC, _S), f32),
        grid=(_B // _G,),
        in_specs=[
            pl.BlockSpec((_G, _KROWS, _S), lambda g: (g, 0, 0)),
            pl.BlockSpec((_OC, _KROWS), lambda g: (0, 0)),
        ],
        out_specs=pl.BlockSpec((_G, _OC, _S), lambda g: (g, 0, 0)),
        compiler_params=pltpu.CompilerParams(
            dimension_semantics=("parallel",),
            vmem_limit_bytes=48 * 1024 * 1024),
        cost_estimate=pl.CostEstimate(
            flops=2 * _B * _OC * _KROWS * _S, transcendentals=0,
            bytes_accessed=4 * (_B * _KROWS * _S + _B * _OC * _S)),
    )(a, wc)

    tk = _K // (2 * _KT)

    part = pl.pallas_call(
        _linear_kernel,
        out_shape=jax.ShapeDtypeStruct((2, _B, 128), f32),
        grid=(2, _KT),
        in_specs=[
            pl.BlockSpec((_B, _OCC, _S), lambda n, k: (0, n * _KT + k, 0)),
            pl.BlockSpec((128, tk), lambda n, k: (0, n * _KT + k)),
            pl.BlockSpec((128, tk), lambda n, k: (0, n * _KT + k)),
        ],
        out_specs=pl.BlockSpec((1, _B, 128), lambda n, k: (n, 0, 0)),
        scratch_shapes=[pltpu.VMEM((_B, 128), f32)],
        compiler_params=pltpu.CompilerParams(
            dimension_semantics=("parallel", "arbitrary"),
            vmem_limit_bytes=48 * 1024 * 1024),
        cost_estimate=pl.CostEstimate(
            flops=2 * _B * 128 * _K + _K * 128, transcendentals=0,
            bytes_accessed=4 * (_B * _K + 2 * 128 * _K + 2 * _B * 128)),
    )(h, lin_mu, lin_stdev_xi)

    bias = (lin_b_mu + lin_b_stdev_xi).reshape(1, 128)
    return part[0] + part[1] + bias


# speedup vs baseline: 1.4108x; 1.1605x over previous
import jax
import jax.numpy as jnp
from jax import lax
from jax.experimental import pallas as pl
from jax.experimental.pallas import tpu as pltpu

_B, _C, _H, _W = 128, 3, 32, 32
_OC, _KH, _KW = 64, 3, 3
_S = _H * _W
_K = _OC * _S
_KROWS = 32
_G = 32
_KT = 4
_OCC = _OC // (2 * _KT)


def _conv_kernel(a_ref, wc_ref, o_ref):
    wc = wc_ref[...]
    for i in range(_G):
        o_ref[i] = jnp.dot(wc, a_ref[i],
                           preferred_element_type=jnp.float32
                           ).astype(o_ref.dtype)


def _linear_kernel(h_ref, wmu_ref, wxi_ref, o_ref, acc_ref):
    k = pl.program_id(1)

    @pl.when(k == 0)
    def _():
        acc_ref[...] = jnp.zeros_like(acc_ref)

    w = wmu_ref[...] + wxi_ref[...]
    acc = acc_ref[...]
    for c in range(_OCC):
        acc += lax.dot_general(
            h_ref[:, c, :], w[:, c * _S:(c + 1) * _S],
            (((1,), (1,)), ((), ())),
            preferred_element_type=jnp.float32)
    acc_ref[...] = acc

    @pl.when(k == _KT - 1)
    def _():
        o_ref[0] = acc_ref[...]


def _build_patches(x):
    bf16 = jnp.bfloat16
    xv = jnp.pad(x.astype(bf16), ((0, 0), (0, 0), (1, 1), (0, 0)))
    pieces = []
    for kh in range(_KH):
        rows = xv[:, :, kh:kh + _H, :]
        for kw in range(_KW):
            if kw == 0:
                sh = jnp.pad(rows, ((0, 0), (0, 0), (0, 0), (1, 0)))[..., :_W]
            elif kw == 1:
                sh = rows
            else:
                sh = jnp.pad(rows, ((0, 0), (0, 0), (0, 0), (0, 1)))[..., 1:]
            pieces.append(sh)
    a = jnp.stack(pieces, axis=1).reshape(_B, 9 * _C, _S)
    pad = jnp.concatenate(
        [jnp.ones((_B, 1, _S), bf16), jnp.zeros((_B, _KROWS - 9 * _C - 1, _S), bf16)],
        axis=1)
    return jnp.concatenate([a, pad], axis=1)


def kernel(x, conv_mu, conv_b_mu, conv_stdev_xi, conv_b_stdev_xi,
           lin_mu, lin_b_mu, lin_stdev_xi, lin_b_stdev_xi):
    f32 = jnp.float32
    a = _build_patches(x)

    wc = (conv_mu + conv_stdev_xi).astype(f32)
    wc = wc.transpose(0, 2, 3, 1).reshape(_OC, 9 * _C)
    cb = (conv_b_mu + conv_b_stdev_xi).astype(f32).reshape(_OC, 1)
    wc = jnp.concatenate(
        [wc, cb, jnp.zeros((_OC, _KROWS - 9 * _C - 1), f32)], axis=1
    ).astype(jnp.bfloat16)

    h = pl.pallas_call(
        _conv_kernel,
        out_shape=jax.ShapeDtypeStruct((_B, _OC, _S), f32),
        grid=(_B // _G,),
        in_specs=[
            pl.BlockSpec((_G, _KROWS, _S), lambda g: (g, 0, 0)),
            pl.BlockSpec((_OC, _KROWS), lambda g: (0, 0)),
        ],
        out_specs=pl.BlockSpec((_G, _OC, _S), lambda g: (g, 0, 0)),
        compiler_params=pltpu.CompilerParams(
            dimension_semantics=("parallel",),
            vmem_limit_bytes=48 * 1024 * 1024),
        cost_estimate=pl.CostEstimate(
            flops=2 * _B * _OC * _KROWS * _S, transcendentals=0,
            bytes_accessed=4 * (_B * _KROWS * _S + _B * _OC * _S)),
    )(a, wc)

    tk = _K // (2 * _KT)

    part = pl.pallas_call(
        _linear_kernel,
        out_shape=jax.ShapeDtypeStruct((2, _B, 128), f32),
        grid=(2, _KT),
        in_specs=[
            pl.BlockSpec((_B, _OCC, _S), lambda n, k: (0, n * _KT + k, 0)),
            pl.BlockSpec((128, tk), lambda n, k: (0, n * _KT + k)),
            pl.BlockSpec((128, tk), lambda n, k: (0, n * _KT + k)),
        ],
        out_specs=pl.BlockSpec((1, _B, 128), lambda n, k: (n, 0, 0)),
        scratch_shapes=[pltpu.VMEM((_B, 128), f32)],
        compiler_params=pltpu.CompilerParams(
            dimension_semantics=("parallel", "arbitrary"),
            vmem_limit_bytes=48 * 1024 * 1024),
        cost_estimate=pl.CostEstimate(
            flops=2 * _B * 128 * _K + _K * 128, transcendentals=0,
            bytes_accessed=4 * (_B * _K + 2 * 128 * _K + 2 * _B * 128)),
    )(h, lin_mu, lin_stdev_xi)

    bias = (lin_b_mu + lin_b_stdev_xi).reshape(1, 128)
    return part[0] + part[1] + bias


# speedup vs baseline: 1.4485x; 1.0267x over previous
import jax
import jax.numpy as jnp
from jax import lax
from jax.experimental import pallas as pl
from jax.experimental.pallas import tpu as pltpu

_B, _C, _H, _W = 128, 3, 32, 32
_OC, _KH, _KW = 64, 3, 3
_S = _H * _W
_K = _OC * _S
_KROWS = 32
_G = 32
_KT = 4
_OCC = _OC // (2 * _KT)


def _conv_kernel(a_ref, wc_ref, o_ref):
    wc = wc_ref[...]
    for i in range(_G):
        o_ref[i] = jnp.dot(wc, a_ref[i],
                           preferred_element_type=jnp.float32
                           ).astype(o_ref.dtype)


def _linear_kernel(h_ref, wmu_ref, wxi_ref, o_ref, acc_ref):
    k = pl.program_id(1)

    @pl.when(k == 0)
    def _():
        acc_ref[...] = jnp.zeros_like(acc_ref)

    w = (wmu_ref[...] + wxi_ref[...]).astype(jnp.bfloat16)
    acc = acc_ref[...]
    for c in range(_OCC):
        acc += lax.dot_general(
            h_ref[:, c, :], w[:, c * _S:(c + 1) * _S],
            (((1,), (1,)), ((), ())),
            preferred_element_type=jnp.float32)
    acc_ref[...] = acc

    @pl.when(k == _KT - 1)
    def _():
        o_ref[0] = acc_ref[...]


def _build_patches(x):
    bf16 = jnp.bfloat16
    xv = jnp.pad(x.astype(bf16), ((0, 0), (0, 0), (1, 1), (0, 0)))
    pieces = []
    for kh in range(_KH):
        rows = xv[:, :, kh:kh + _H, :]
        for kw in range(_KW):
            if kw == 0:
                sh = jnp.pad(rows, ((0, 0), (0, 0), (0, 0), (1, 0)))[..., :_W]
            elif kw == 1:
                sh = rows
            else:
                sh = jnp.pad(rows, ((0, 0), (0, 0), (0, 0), (0, 1)))[..., 1:]
            pieces.append(sh)
    a = jnp.stack(pieces, axis=1).reshape(_B, 9 * _C, _S)
    pad = jnp.concatenate(
        [jnp.ones((_B, 1, _S), bf16), jnp.zeros((_B, _KROWS - 9 * _C - 1, _S), bf16)],
        axis=1)
    return jnp.concatenate([a, pad], axis=1)


def kernel(x, conv_mu, conv_b_mu, conv_stdev_xi, conv_b_stdev_xi,
           lin_mu, lin_b_mu, lin_stdev_xi, lin_b_stdev_xi):
    f32 = jnp.float32
    a = _build_patches(x)

    wc = (conv_mu + conv_stdev_xi).astype(f32)
    wc = wc.transpose(0, 2, 3, 1).reshape(_OC, 9 * _C)
    cb = (conv_b_mu + conv_b_stdev_xi).astype(f32).reshape(_OC, 1)
    wc = jnp.concatenate(
        [wc, cb, jnp.zeros((_OC, _KROWS - 9 * _C - 1), f32)], axis=1
    ).astype(jnp.bfloat16)

    h = pl.pallas_call(
        _conv_kernel,
        out_shape=jax.ShapeDtypeStruct((_B, _OC, _S), jnp.bfloat16),
        grid=(_B // _G,),
        in_specs=[
            pl.BlockSpec((_G, _KROWS, _S), lambda g: (g, 0, 0)),
            pl.BlockSpec((_OC, _KROWS), lambda g: (0, 0)),
        ],
        out_specs=pl.BlockSpec((_G, _OC, _S), lambda g: (g, 0, 0)),
        compiler_params=pltpu.CompilerParams(
            dimension_semantics=("parallel",),
            vmem_limit_bytes=48 * 1024 * 1024),
        cost_estimate=pl.CostEstimate(
            flops=2 * _B * _OC * _KROWS * _S, transcendentals=0,
            bytes_accessed=4 * (_B * _KROWS * _S + _B * _OC * _S)),
    )(a, wc)

    tk = _K // (2 * _KT)

    part = pl.pallas_call(
        _linear_kernel,
        out_shape=jax.ShapeDtypeStruct((2, _B, 128), f32),
        grid=(2, _KT),
        in_specs=[
            pl.BlockSpec((_B, _OCC, _S), lambda n, k: (0, n * _KT + k, 0)),
            pl.BlockSpec((128, tk), lambda n, k: (0, n * _KT + k)),
            pl.BlockSpec((128, tk), lambda n, k: (0, n * _KT + k)),
        ],
        out_specs=pl.BlockSpec((1, _B, 128), lambda n, k: (n, 0, 0)),
        scratch_shapes=[pltpu.VMEM((_B, 128), f32)],
        compiler_params=pltpu.CompilerParams(
            dimension_semantics=("parallel", "arbitrary"),
            vmem_limit_bytes=48 * 1024 * 1024),
        cost_estimate=pl.CostEstimate(
            flops=2 * _B * 128 * _K + _K * 128, transcendentals=0,
            bytes_accessed=4 * (_B * _K + 2 * 128 * _K + 2 * _B * 128)),
    )(h, lin_mu, lin_stdev_xi)

    bias = (lin_b_mu + lin_b_stdev_xi).reshape(1, 128)
    return part[0] + part[1] + bias


# speedup vs baseline: 1.4494x; 1.0006x over previous
import jax
import jax.numpy as jnp
from jax import lax
from jax.experimental import pallas as pl
from jax.experimental.pallas import tpu as pltpu

_B, _C, _H, _W = 128, 3, 32, 32
_OC, _KH, _KW = 64, 3, 3
_S = _H * _W
_K = _OC * _S
_KROWS = 32
_G = 32
_KT = 2
_OCC = _OC // (2 * _KT)


def _conv_kernel(a_ref, wc_ref, o_ref):
    wc = wc_ref[...]
    for i in range(_G):
        o_ref[i] = jnp.dot(wc, a_ref[i],
                           preferred_element_type=jnp.float32
                           ).astype(o_ref.dtype)


def _linear_kernel(h_ref, wmu_ref, wxi_ref, o_ref, acc_ref):
    k = pl.program_id(1)

    @pl.when(k == 0)
    def _():
        acc_ref[...] = jnp.zeros_like(acc_ref)

    w = (wmu_ref[...] + wxi_ref[...]).astype(jnp.bfloat16)
    acc = acc_ref[...]
    for c in range(_OCC):
        acc += lax.dot_general(
            h_ref[:, c, :], w[:, c * _S:(c + 1) * _S],
            (((1,), (1,)), ((), ())),
            preferred_element_type=jnp.float32)
    acc_ref[...] = acc

    @pl.when(k == _KT - 1)
    def _():
        o_ref[0] = acc_ref[...]


def _build_patches(x):
    bf16 = jnp.bfloat16
    xv = jnp.pad(x.astype(bf16), ((0, 0), (0, 0), (1, 1), (0, 0)))
    pieces = []
    for kh in range(_KH):
        rows = xv[:, :, kh:kh + _H, :]
        for kw in range(_KW):
            if kw == 0:
                sh = jnp.pad(rows, ((0, 0), (0, 0), (0, 0), (1, 0)))[..., :_W]
            elif kw == 1:
                sh = rows
            else:
                sh = jnp.pad(rows, ((0, 0), (0, 0), (0, 0), (0, 1)))[..., 1:]
            pieces.append(sh)
    a = jnp.stack(pieces, axis=1).reshape(_B, 9 * _C, _S)
    pad = jnp.concatenate(
        [jnp.ones((_B, 1, _S), bf16), jnp.zeros((_B, _KROWS - 9 * _C - 1, _S), bf16)],
        axis=1)
    return jnp.concatenate([a, pad], axis=1)


def kernel(x, conv_mu, conv_b_mu, conv_stdev_xi, conv_b_stdev_xi,
           lin_mu, lin_b_mu, lin_stdev_xi, lin_b_stdev_xi):
    f32 = jnp.float32
    a = _build_patches(x)

    wc = (conv_mu + conv_stdev_xi).astype(f32)
    wc = wc.transpose(0, 2, 3, 1).reshape(_OC, 9 * _C)
    cb = (conv_b_mu + conv_b_stdev_xi).astype(f32).reshape(_OC, 1)
    wc = jnp.concatenate(
        [wc, cb, jnp.zeros((_OC, _KROWS - 9 * _C - 1), f32)], axis=1
    ).astype(jnp.bfloat16)

    h = pl.pallas_call(
        _conv_kernel,
        out_shape=jax.ShapeDtypeStruct((_B, _OC, _S), jnp.bfloat16),
        grid=(_B // _G,),
        in_specs=[
            pl.BlockSpec((_G, _KROWS, _S), lambda g: (g, 0, 0)),
            pl.BlockSpec((_OC, _KROWS), lambda g: (0, 0)),
        ],
        out_specs=pl.BlockSpec((_G, _OC, _S), lambda g: (g, 0, 0)),
        compiler_params=pltpu.CompilerParams(
            dimension_semantics=("parallel",),
            vmem_limit_bytes=48 * 1024 * 1024),
        cost_estimate=pl.CostEstimate(
            flops=2 * _B * _OC * _KROWS * _S, transcendentals=0,
            bytes_accessed=4 * (_B * _KROWS * _S + _B * _OC * _S)),
    )(a, wc)

    tk = _K // (2 * _KT)

    part = pl.pallas_call(
        _linear_kernel,
        out_shape=jax.ShapeDtypeStruct((2, _B, 128), f32),
        grid=(2, _KT),
        in_specs=[
            pl.BlockSpec((_B, _OCC, _S), lambda n, k: (0, n * _KT + k, 0)),
            pl.BlockSpec((128, tk), lambda n, k: (0, n * _KT + k)),
            pl.BlockSpec((128, tk), lambda n, k: (0, n * _KT + k)),
        ],
        out_specs=pl.BlockSpec((1, _B, 128), lambda n, k: (n, 0, 0)),
        scratch_shapes=[pltpu.VMEM((_B, 128), f32)],
        compiler_params=pltpu.CompilerParams(
            dimension_semantics=("parallel", "arbitrary"),
            vmem_limit_bytes=48 * 1024 * 1024),
        cost_estimate=pl.CostEstimate(
            flops=2 * _B * 128 * _K + _K * 128, transcendentals=0,
            bytes_accessed=4 * (_B * _K + 2 * 128 * _K + 2 * _B * 128)),
    )(h, lin_mu, lin_stdev_xi)

    bias = (lin_b_mu + lin_b_stdev_xi).reshape(1, 128)
    return part[0] + part[1] + bias
